# baseline (speedup 1.0000x reference)
import numpy as np
import jax
import jax.numpy as jnp
from jax.experimental import pallas as pl
from jax.experimental.pallas import tpu as pltpu

_BT = 256

_EV1 = np.concatenate([np.arange(w * 32, w * 32 + 32) for w in range(0, 28, 2)])
_OD1 = np.concatenate([np.arange(w * 32, w * 32 + 32) for w in range(1, 28, 2)])
_EV2 = np.concatenate([np.arange(w * 64, w * 64 + 64) for w in range(0, 12, 2)])
_OD2 = np.concatenate([np.arange(w * 64, w * 64 + 64) for w in range(1, 12, 2)])


def _fused_kernel(x_ref, b1_ref, b2_ref, s2_ref,
                  w1_ref, c1_ref, w2_ref, c2_ref, w3_ref, c3_ref, o_ref):
    bt = x_ref.shape[1]
    x = x_ref[...]

    zrow = jnp.zeros((1, bt, 28), jnp.float32)
    top = jnp.concatenate([zrow, x[:27]], axis=0)
    bot = jnp.concatenate([x[1:], zrow], axis=0)
    ones = jnp.ones((28, bt, 1), jnp.float32)
    lhs1 = jnp.concatenate([top, x, bot, ones], axis=2).reshape(28 * bt, 85)
    y1 = jnp.dot(lhs1, b1_ref[...], preferred_element_type=jnp.float32)

    yh = jnp.max(y1.reshape(14, 2, bt, 1024), axis=1)
    p1 = jnp.maximum(jnp.maximum(yh[..., :512], yh[..., 512:]), 0.0)

    h = c1_ref[...]
    for i in range(6):
        lhs2 = jnp.concatenate([p1[2 * i:2 * i + 2], p1[2 * i + 1:2 * i + 3],
                                p1[2 * i + 2:2 * i + 4]], axis=2)
        z = jnp.dot(lhs2.reshape(2 * bt, 1536), b2_ref[...],
                    preferred_element_type=jnp.float32)
        zh = jnp.max(z.reshape(2, bt, 768), axis=0) + s2_ref[...]
        q = jnp.maximum(jnp.maximum(zh[:, :384], zh[:, 384:]), 0.0)
        h = h + jnp.dot(q, w1_ref[384 * i:384 * (i + 1)],
                        preferred_element_type=jnp.float32)

    h = jnp.dot(h, w2_ref[...], preferred_element_type=jnp.float32) + c2_ref[...]
    h = jnp.dot(h, w3_ref[...], preferred_element_type=jnp.float32) + c3_ref[...]
    m = jnp.max(h, axis=1, keepdims=True)
    s = h - m
    lse = jnp.log(jnp.sum(jnp.exp(s), axis=1, keepdims=True))
    o_ref[...] = (s - lse).astype(o_ref.dtype)


def kernel(x, B1, sh1t, B2, sh2t, fw1, fb1, fw2, fb2, fw3, fb3):
    n = x.shape[0]
    bt = _BT if n % _BT == 0 else 8
    xt = jnp.transpose(x.reshape(n, 28, 28), (1, 0, 2))

    B1p = jnp.concatenate([B1[:, _EV1], jnp.zeros((84, 64), B1.dtype),
                           B1[:, _OD1], jnp.zeros((84, 64), B1.dtype)], axis=1)
    s1p = jnp.concatenate([sh1t[:, _EV1], jnp.zeros((1, 64), sh1t.dtype),
                           sh1t[:, _OD1], jnp.zeros((1, 64), sh1t.dtype)], axis=1)
    B1p = jnp.concatenate([B1p, s1p], axis=0)
    z64 = jnp.zeros((64, 768), B2.dtype)
    B2r = jnp.concatenate([B2[0:448], z64, B2[448:896], z64, B2[896:1344], z64],
                          axis=0)
    B2p = jnp.concatenate([B2r[:, _EV2], B2r[:, _OD2]], axis=1)
    s2p = jnp.concatenate([sh2t[:, _EV2], sh2t[:, _OD2]], axis=1)

    return pl.pallas_call(
        _fused_kernel,
        out_shape=jax.ShapeDtypeStruct((n, 10), jnp.float32),
        grid=(n // bt,),
        in_specs=[
            pl.BlockSpec((28, bt, 28), lambda i: (0, i, 0)),
            pl.BlockSpec((85, 1024), lambda i: (0, 0)),
            pl.BlockSpec((1536, 768), lambda i: (0, 0)),
            pl.BlockSpec((1, 768), lambda i: (0, 0)),
            pl.BlockSpec((2304, 640), lambda i: (0, 0)),
            pl.BlockSpec((1, 640), lambda i: (0, 0)),
            pl.BlockSpec((640, 128), lambda i: (0, 0)),
            pl.BlockSpec((1, 128), lambda i: (0, 0)),
            pl.BlockSpec((128, 10), lambda i: (0, 0)),
            pl.BlockSpec((1, 10), lambda i: (0, 0)),
        ],
        out_specs=pl.BlockSpec((bt, 10), lambda i: (i, 0)),
        compiler_params=pltpu.CompilerParams(
            dimension_semantics=("parallel",),
            vmem_limit_bytes=64 * 1024 * 1024),
    )(xt, B1p, B2p, s2p, fw1, fb1, fw2, fb2, fw3, fb3)


# speedup vs baseline: 1.0185x; 1.0185x over previous
import numpy as np
import jax
import jax.numpy as jnp
from jax.experimental import pallas as pl
from jax.experimental.pallas import tpu as pltpu

_BT = 256

_EV1 = np.concatenate([np.arange(w * 32, w * 32 + 32) for w in range(0, 28, 2)])
_OD1 = np.concatenate([np.arange(w * 32, w * 32 + 32) for w in range(1, 28, 2)])
_EV2 = np.concatenate([np.arange(w * 64, w * 64 + 64) for w in range(0, 12, 2)])
_OD2 = np.concatenate([np.arange(w * 64, w * 64 + 64) for w in range(1, 12, 2)])


def _fused_kernel(x_ref, b1_ref, b2_ref, s2_ref,
                  w1_ref, c1_ref, w2_ref, c2_ref, w3_ref, c3_ref, o_ref):
    bt = x_ref.shape[1]
    x = x_ref[...]

    zrow = jnp.zeros((1, bt, 28), jnp.float32)
    top = jnp.concatenate([zrow, x[:27]], axis=0)
    bot = jnp.concatenate([x[1:], zrow], axis=0)
    ones = jnp.ones((28, bt, 1), jnp.float32)
    lhs1 = jnp.concatenate([top, x, bot, ones], axis=2).reshape(28 * bt, 85)
    y1 = jnp.dot(lhs1, b1_ref[...], preferred_element_type=jnp.float32)

    yh = jnp.max(y1.reshape(14, 2, bt, 1024), axis=1)
    p1 = jnp.maximum(jnp.maximum(yh[..., :512], yh[..., 512:]), 0.0)

    lhs2 = jnp.concatenate([p1[0:12], p1[1:13], p1[2:14]], axis=2)
    z = jnp.dot(lhs2.reshape(12 * bt, 1536), b2_ref[...],
                preferred_element_type=jnp.float32)

    zh = jnp.max(z.reshape(6, 2, bt, 768), axis=1) + s2_ref[...]
    q = jnp.maximum(jnp.maximum(zh[..., :384], zh[..., 384:]), 0.0)

    h = c1_ref[...] + jnp.dot(q[0], w1_ref[0:384],
                              preferred_element_type=jnp.float32)
    for i in range(1, 6):
        h = h + jnp.dot(q[i], w1_ref[384 * i:384 * (i + 1)],
                        preferred_element_type=jnp.float32)
    h = jnp.dot(h, w2_ref[...], preferred_element_type=jnp.float32) + c2_ref[...]
    h = jnp.dot(h, w3_ref[...], preferred_element_type=jnp.float32) + c3_ref[...]
    m = jnp.max(h, axis=1, keepdims=True)
    s = h - m
    lse = jnp.log(jnp.sum(jnp.exp(s), axis=1, keepdims=True))
    o_ref[...] = (s - lse).astype(o_ref.dtype)


def kernel(x, B1, sh1t, B2, sh2t, fw1, fb1, fw2, fb2, fw3, fb3):
    n = x.shape[0]
    bt = _BT if n % _BT == 0 else 8
    xt = jnp.transpose(x.reshape(n, 28, 28), (1, 0, 2))

    B1p = jnp.concatenate([B1[:, _EV1], jnp.zeros((84, 64), B1.dtype),
                           B1[:, _OD1], jnp.zeros((84, 64), B1.dtype)], axis=1)
    s1p = jnp.concatenate([sh1t[:, _EV1], jnp.zeros((1, 64), sh1t.dtype),
                           sh1t[:, _OD1], jnp.zeros((1, 64), sh1t.dtype)], axis=1)
    B1p = jnp.concatenate([B1p, s1p], axis=0)
    z64 = jnp.zeros((64, 768), B2.dtype)
    B2r = jnp.concatenate([B2[0:448], z64, B2[448:896], z64, B2[896:1344], z64],
                          axis=0)
    B2p = jnp.concatenate([B2r[:, _EV2], B2r[:, _OD2]], axis=1)
    s2p = jnp.concatenate([sh2t[:, _EV2], sh2t[:, _OD2]], axis=1)

    return pl.pallas_call(
        _fused_kernel,
        out_shape=jax.ShapeDtypeStruct((n, 10), jnp.float32),
        grid=(n // bt,),
        in_specs=[
            pl.BlockSpec((28, bt, 28), lambda i: (0, i, 0)),
            pl.BlockSpec((85, 1024), lambda i: (0, 0)),
            pl.BlockSpec((1536, 768), lambda i: (0, 0)),
            pl.BlockSpec((1, 768), lambda i: (0, 0)),
            pl.BlockSpec((2304, 640), lambda i: (0, 0)),
            pl.BlockSpec((1, 640), lambda i: (0, 0)),
            pl.BlockSpec((640, 128), lambda i: (0, 0)),
            pl.BlockSpec((1, 128), lambda i: (0, 0)),
            pl.BlockSpec((128, 10), lambda i: (0, 0)),
            pl.BlockSpec((1, 10), lambda i: (0, 0)),
        ],
        out_specs=pl.BlockSpec((bt, 10), lambda i: (i, 0)),
        compiler_params=pltpu.CompilerParams(
            dimension_semantics=("parallel",),
            vmem_limit_bytes=64 * 1024 * 1024),
    )(xt, B1p, B2p, s2p, fw1, fb1, fw2, fb2, fw3, fb3)
